# Initial kernel scaffold; baseline (speedup 1.0000x reference)
import jax
import jax.numpy as jnp
from jax.experimental import pallas as pl


def kernel(x, Wa, Wh, Wqkv, Wout, Wgate, Wproj_in, conv_w, conv_b, Wproj_out, Wg_moe, w1, w2, w3, g1, b1, g2, b2, ls1g, ls2g):
    raise NotImplementedError("write your pallas kernel here")


# speedup vs baseline: 1.1608x; 1.1608x over previous
import functools

import jax
import jax.numpy as jnp
from jax import lax
from jax.experimental import pallas as pl

H = 16
LN_EPS = 1e-5
_INTERPRET = False


def _pcall(*args, **kwargs):
    return pl.pallas_call(*args, interpret=_INTERPRET, **kwargs)


def _ln(x, g, b):
    m = x.mean(-1, keepdims=True)
    v = ((x - m) ** 2).mean(-1, keepdims=True)
    return (x - m) / jnp.sqrt(v + LN_EPS) * g + b


def _dot(a, b):
    return lax.dot_general(a, b, (((1,), (0,)), ((), ())),
                           preferred_element_type=jnp.float32)


def _k1(x_ref, w_ref, g_ref, b_ref, o_ref):
    n = _ln(x_ref[...], g_ref[...], b_ref[...])
    o_ref[...] = _dot(n.astype(jnp.bfloat16), w_ref[...]).astype(jnp.bfloat16)


def _k2(q_ref, k_ref, v_ref, o_ref, *, scale):
    s = lax.dot_general(q_ref[0], k_ref[0], (((1,), (1,)), ((), ())),
                        preferred_element_type=jnp.float32) * scale
    s = s - jnp.max(s, axis=-1, keepdims=True)
    p = jnp.exp(s)
    p = p / jnp.sum(p, axis=-1, keepdims=True)
    o_ref[0] = _dot(p.astype(jnp.bfloat16), v_ref[0]).astype(jnp.bfloat16)


def _k3(xs_ref, z_ref, w_ref, cb_ref, o_ref):
    x = xs_ref[...].astype(jnp.float32)
    w = w_ref[...]
    acc = x * w[3:4, :]
    for k in (1, 2, 3):
        shifted = jnp.concatenate(
            [jnp.zeros((k, x.shape[1]), jnp.float32), x[:-k, :]], axis=0)
        acc = acc + shifted * w[3 - k:4 - k, :]
    acc = acc + cb_ref[...]
    z = z_ref[...].astype(jnp.float32)
    o_ref[...] = (acc * jax.nn.sigmoid(acc) * jax.nn.sigmoid(z)
                  ).astype(jnp.bfloat16)


def _k4(x_ref, mg_ref, h_ref, gp_ref, al_ref, wo_ref, wp_ref, wg_ref,
        g2_ref, b2_ref, ls1_ref, s_ref, y2_ref, lg_ref):
    a_out = _dot(mg_ref[...], wo_ref[...])
    a_out = a_out * jax.nn.sigmoid(gp_ref[...].astype(jnp.float32))
    s_out = _dot(h_ref[...], wp_ref[...])
    alpha = jax.nn.sigmoid(al_ref[...][:, 0:1].astype(jnp.float32))
    blended = ((1.0 - alpha) * s_out + alpha * a_out) * ls1_ref[...]
    x = x_ref[...]
    y = x + blended
    s_ref[...] = x + y
    y2 = _ln(y, g2_ref[...], b2_ref[...]).astype(jnp.bfloat16)
    y2_ref[...] = y2
    lg_ref[...] = _dot(y2, wg_ref[...])


def _k5(y2_ref, lg_ref, s_ref, w1_ref, w3_ref, w2_ref, ls2_ref, o_ref, *,
        n_e):
    e = pl.program_id(1)
    k = pl.program_id(2)
    f = y2_ref[...]
    t1 = _dot(f, w1_ref[0])
    t1 = t1 * jax.nn.sigmoid(t1)
    t = (t1 * _dot(f, w3_ref[0])).astype(jnp.bfloat16)
    part = _dot(t, w2_ref[0])

    lg = lg_ref[...]
    mx = jnp.max(lg, axis=-1, keepdims=True)
    ex = jnp.exp(lg - mx)
    probs = ex / jnp.sum(ex, axis=-1, keepdims=True)
    iota = lax.broadcasted_iota(jnp.int32, probs.shape, 1)
    m1 = jnp.max(probs, axis=-1, keepdims=True)
    i1 = jnp.min(jnp.where(probs == m1, iota, n_e), axis=-1, keepdims=True)
    p2 = jnp.where(iota == i1, -jnp.inf, probs)
    m2 = jnp.max(p2, axis=-1, keepdims=True)
    i2 = jnp.min(jnp.where(p2 == m2, iota, n_e), axis=-1, keepdims=True)
    we = (jnp.where(i1 == e, m1, 0.0) + jnp.where(i2 == e, m2, 0.0)
          ) / (m1 + m2 + 1e-8)
    contrib = (we * part) * ls2_ref[...]

    @pl.when((e == 0) & (k == 0))
    def _():
        o_ref[...] = s_ref[...] + contrib

    @pl.when((e > 0) | (k > 0))
    def _():
        o_ref[...] += contrib


def kernel(x, Wa, Wh, Wqkv, Wout, Wgate, Wproj_in, conv_w, conv_b, Wproj_out,
           Wg_moe, w1, w2, w3, g1, b1, g2, b2, ls1g, ls2g):
    del Wh
    B, L, D = x.shape
    DH = D // H
    E, _, DFF = w1.shape
    bf = jnp.bfloat16

    LT = 256
    MT = 512
    FT = 512
    xf = x.reshape(L, D)

    wcat = jnp.concatenate(
        [Wqkv, Wproj_in, Wgate,
         jnp.pad(Wa, ((0, 0), (0, 127))), jnp.zeros((D, 128), jnp.float32)],
        axis=1).astype(bf)
    NW = 3 * D + 2 * D + D + 256
    proj = _pcall(
        _k1,
        grid=(L // LT,),
        in_specs=[
            pl.BlockSpec((LT, D), lambda i: (i, 0)),
            pl.BlockSpec((D, NW), lambda i: (0, 0)),
            pl.BlockSpec((1, D), lambda i: (0, 0)),
            pl.BlockSpec((1, D), lambda i: (0, 0)),
        ],
        out_specs=pl.BlockSpec((LT, NW), lambda i: (i, 0)),
        out_shape=jax.ShapeDtypeStruct((L, NW), bf),
    )(xf, wcat, g1.reshape(1, D), b1.reshape(1, D))

    qh = proj[:, 0 * D:1 * D].reshape(L, H, DH).transpose(1, 0, 2)
    kh = proj[:, 1 * D:2 * D].reshape(L, H, DH).transpose(1, 0, 2)
    vh = proj[:, 2 * D:3 * D].reshape(L, H, DH).transpose(1, 0, 2)
    xs = proj[:, 3 * D:4 * D]
    zg = proj[:, 4 * D:5 * D]
    gate_pre = proj[:, 5 * D:6 * D]
    alpha_col = proj[:, 6 * D:6 * D + 128]

    attn = _pcall(
        functools.partial(_k2, scale=DH ** -0.5),
        grid=(H, L // LT),
        in_specs=[
            pl.BlockSpec((1, LT, DH), lambda h, i: (h, i, 0)),
            pl.BlockSpec((1, L, DH), lambda h, i: (h, 0, 0)),
            pl.BlockSpec((1, L, DH), lambda h, i: (h, 0, 0)),
        ],
        out_specs=pl.BlockSpec((1, LT, DH), lambda h, i: (h, i, 0)),
        out_shape=jax.ShapeDtypeStruct((H, L, DH), bf),
    )(qh, kh, vh)
    merged = attn.transpose(1, 0, 2).reshape(L, D)

    CT = 256
    cwp = jnp.pad(conv_w[:, 0, :].T, ((0, 4), (0, 0)))
    hconv = _pcall(
        _k3,
        grid=(D // CT,),
        in_specs=[
            pl.BlockSpec((L, CT), lambda i: (0, i)),
            pl.BlockSpec((L, CT), lambda i: (0, i)),
            pl.BlockSpec((8, CT), lambda i: (0, i)),
            pl.BlockSpec((1, CT), lambda i: (0, i)),
        ],
        out_specs=pl.BlockSpec((L, CT), lambda i: (0, i)),
        out_shape=jax.ShapeDtypeStruct((L, D), bf),
    )(xs, zg, cwp, conv_b.reshape(1, D))

    wg_pad = jnp.pad(Wg_moe, ((0, 0), (0, 128 - E))).astype(bf)
    s_res, y2, logits = _pcall(
        _k4,
        grid=(L // LT,),
        in_specs=[
            pl.BlockSpec((LT, D), lambda i: (i, 0)),
            pl.BlockSpec((LT, D), lambda i: (i, 0)),
            pl.BlockSpec((LT, D), lambda i: (i, 0)),
            pl.BlockSpec((LT, D), lambda i: (i, 0)),
            pl.BlockSpec((LT, 128), lambda i: (i, 0)),
            pl.BlockSpec((D, D), lambda i: (0, 0)),
            pl.BlockSpec((D, D), lambda i: (0, 0)),
            pl.BlockSpec((D, 128), lambda i: (0, 0)),
            pl.BlockSpec((1, D), lambda i: (0, 0)),
            pl.BlockSpec((1, D), lambda i: (0, 0)),
            pl.BlockSpec((1, D), lambda i: (0, 0)),
        ],
        out_specs=[
            pl.BlockSpec((LT, D), lambda i: (i, 0)),
            pl.BlockSpec((LT, D), lambda i: (i, 0)),
            pl.BlockSpec((LT, 128), lambda i: (i, 0)),
        ],
        out_shape=[
            jax.ShapeDtypeStruct((L, D), jnp.float32),
            jax.ShapeDtypeStruct((L, D), bf),
            jax.ShapeDtypeStruct((L, 128), jnp.float32),
        ],
    )(xf, merged, hconv, gate_pre, alpha_col, Wout.astype(bf),
      Wproj_out.astype(bf), wg_pad, g2.reshape(1, D), b2.reshape(1, D),
      ls1g.reshape(1, D))
    logits8 = logits[:, :E]

    out = _pcall(
        functools.partial(_k5, n_e=E),
        grid=(L // MT, E, DFF // FT),
        in_specs=[
            pl.BlockSpec((MT, D), lambda m, e, k: (m, 0)),
            pl.BlockSpec((MT, E), lambda m, e, k: (m, 0)),
            pl.BlockSpec((MT, D), lambda m, e, k: (m, 0)),
            pl.BlockSpec((1, D, FT), lambda m, e, k: (e, 0, k)),
            pl.BlockSpec((1, D, FT), lambda m, e, k: (e, 0, k)),
            pl.BlockSpec((1, FT, D), lambda m, e, k: (e, k, 0)),
            pl.BlockSpec((1, D), lambda m, e, k: (0, 0)),
        ],
        out_specs=pl.BlockSpec((MT, D), lambda m, e, k: (m, 0)),
        out_shape=jax.ShapeDtypeStruct((L, D), jnp.float32),
    )(y2, logits8, s_res, w1.astype(bf), w3.astype(bf), w2.astype(bf),
      ls2g.reshape(1, D))

    return out.reshape(B, L, D)
